# baseline (speedup 1.0000x reference)
import functools

import jax
import jax.numpy as jnp
from jax.experimental import pallas as pl
from jax.experimental.pallas import tpu as pltpu

T = 16384
D = 2048
H = 1024
E = 16
K = 2
TM = 512
NSTEPS = T // (2 * TM)
TH = T // 2


def _tile_outputs(x, w1cat, b1, w2, b2, eye):
    xh32 = jax.lax.bitcast_convert_type(
        jax.lax.bitcast_convert_type(x, jnp.uint32) & jnp.uint32(0xFFFF0000),
        jnp.float32)
    xh = xh32.astype(jnp.bfloat16)
    xl = (x - xh32).astype(jnp.bfloat16)
    xcat = jnp.concatenate([xh, xh, xl], axis=1)
    h = jnp.dot(xcat, w1cat, preferred_element_type=jnp.float32)
    h = jnp.maximum(h + b1, 0.0)
    logits = jnp.dot(h, w2, preferred_element_type=jnp.float32)
    logits = logits + b2

    lt = logits.T
    iota = jax.lax.broadcasted_iota(jnp.int32, (E, TM), 0)
    m1 = jnp.max(lt, axis=0, keepdims=True)
    i1 = jnp.min(jnp.where(lt == m1, iota, E), axis=0, keepdims=True)
    masked = jnp.where(iota == i1, -jnp.inf, lt)
    m2 = jnp.max(masked, axis=0, keepdims=True)
    i2 = jnp.min(jnp.where(masked == m2, iota, E), axis=0, keepdims=True)

    et = jnp.exp(lt - m1)
    zt = jnp.sum(et, axis=0, keepdims=True)
    pt = et / zt
    w1p = jnp.max(pt, axis=0, keepdims=True)
    w2p = jnp.max(jnp.where(iota == i1, -1.0, pt), axis=0, keepdims=True)
    denom = jnp.maximum(w1p + w2p, 1e-6)
    wts = jnp.concatenate([w1p, w2p], axis=0) / denom
    idx = jnp.concatenate([i1, i2], axis=0).astype(jnp.float32)

    tr = lambda a: jax.lax.dot_general(
        eye, a, (((1,), (1,)), ((), ())), preferred_element_type=jnp.float32)
    return tr(wts), tr(idx).astype(jnp.int32), tr(pt)


def _router_kernel(xa_ref, xb_ref, w1cat_ref, b1_ref, w2_ref, b2_ref,
                   eye_ref, w_ref, i_ref, p_ref):
    eye = eye_ref[...]
    w1cat = w1cat_ref[...]
    b1 = b1_ref[...]
    w2 = w2_ref[...]
    b2 = b2_ref[...]
    wa, ia, pa = _tile_outputs(xa_ref[...], w1cat, b1, w2, b2, eye)
    wb, ib, pb = _tile_outputs(xb_ref[...], w1cat, b1, w2, b2, eye)
    w_ref[0], w_ref[1] = wa, wb
    i_ref[0], i_ref[1] = ia, ib
    p_ref[0], p_ref[1] = pa, pb


@functools.partial(jax.jit, static_argnames=("interpret",))
def kernel(pooled_feat, W1, b1, W2, b2, interpret=False):
    b1r = b1.reshape(1, H)
    b2r = b2.reshape(1, E)
    eye = jnp.eye(TM, dtype=jnp.float32)
    w1h = W1.astype(jnp.bfloat16)
    w1l = (W1 - w1h.astype(jnp.float32)).astype(jnp.bfloat16)
    w1cat = jnp.concatenate([w1h, w1l, w1h], axis=0)
    grid = (NSTEPS,)
    out = pl.pallas_call(
        _router_kernel,
        grid=grid,
        in_specs=[
            pl.BlockSpec((TM, D), lambda i: (i, 0)),
            pl.BlockSpec((TM, D), lambda i: (i + NSTEPS, 0)),
            pl.BlockSpec((3 * D, H), lambda i: (0, 0)),
            pl.BlockSpec((1, H), lambda i: (0, 0)),
            pl.BlockSpec((H, E), lambda i: (0, 0)),
            pl.BlockSpec((1, E), lambda i: (0, 0)),
            pl.BlockSpec((TM, TM), lambda i: (0, 0)),
        ],
        out_specs=[
            pl.BlockSpec((2, TM, K), lambda i: (0, i, 0)),
            pl.BlockSpec((2, TM, K), lambda i: (0, i, 0)),
            pl.BlockSpec((2, TM, E), lambda i: (0, i, 0)),
        ],
        out_shape=[
            jax.ShapeDtypeStruct((2, TH, K), jnp.float32),
            jax.ShapeDtypeStruct((2, TH, K), jnp.int32),
            jax.ShapeDtypeStruct((2, TH, E), jnp.float32),
        ],
        compiler_params=pltpu.CompilerParams(
            dimension_semantics=("parallel",)),
        interpret=interpret,
    )(pooled_feat, pooled_feat, w1cat, b1r, W2, b2r, eye)
    return (out[0].reshape(T, K), out[1].reshape(T, K), out[2].reshape(T, E))


# speedup vs baseline: 2.6271x; 2.6271x over previous
import functools

import jax
import jax.numpy as jnp
from jax.experimental import pallas as pl
from jax.experimental.pallas import tpu as pltpu

T = 16384
D = 2048
H = 1024
E = 16
K = 2
TM = 1024
NSTEPS = T // TM


def _tile_outputs(x, w1, b1, w2, b2):
    h = jnp.dot(x, w1, preferred_element_type=jnp.float32)
    h = jnp.maximum(h + b1, 0.0)
    logits = jnp.dot(h, w2, preferred_element_type=jnp.float32)
    logits = logits + b2

    lt = logits.T
    iota = jax.lax.broadcasted_iota(jnp.int32, (E, TM), 0)
    m1 = jnp.max(lt, axis=0, keepdims=True)
    i1 = jnp.min(jnp.where(lt == m1, iota, E), axis=0, keepdims=True)
    masked = jnp.where(iota == i1, -jnp.inf, lt)
    m2 = jnp.max(masked, axis=0, keepdims=True)
    i2 = jnp.min(jnp.where(masked == m2, iota, E), axis=0, keepdims=True)

    et = jnp.exp(lt - m1)
    zt = jnp.sum(et, axis=0, keepdims=True)
    pt = et / zt
    w1p = jnp.max(pt, axis=0, keepdims=True)
    w2p = jnp.max(jnp.where(iota == i1, -1.0, pt), axis=0, keepdims=True)
    denom = jnp.maximum(w1p + w2p, 1e-6)
    wts = jnp.concatenate([w1p, w2p], axis=0) / denom
    idx = jnp.concatenate([i1, i2], axis=0).astype(jnp.float32)

    return wts.T, idx.astype(jnp.int32).T, pt.T


def _router_kernel(x_ref, w1_ref, b1_ref, w2_ref, b2_ref,
                   w_ref, i_ref, p_ref):
    wts, idx, pt = _tile_outputs(x_ref[...], w1_ref[...], b1_ref[...],
                                 w2_ref[...], b2_ref[...])
    w_ref[...] = wts
    i_ref[...] = idx
    p_ref[...] = pt


@functools.partial(jax.jit, static_argnames=("interpret",))
def kernel(pooled_feat, W1, b1, W2, b2, interpret=False):
    b1r = b1.reshape(1, H)
    b2r = b2.reshape(1, E)
    grid = (NSTEPS,)
    out = pl.pallas_call(
        _router_kernel,
        grid=grid,
        in_specs=[
            pl.BlockSpec((TM, D), lambda i: (i, 0)),
            pl.BlockSpec((D, H), lambda i: (0, 0)),
            pl.BlockSpec((1, H), lambda i: (0, 0)),
            pl.BlockSpec((H, E), lambda i: (0, 0)),
            pl.BlockSpec((1, E), lambda i: (0, 0)),
        ],
        out_specs=[
            pl.BlockSpec((TM, K), lambda i: (i, 0)),
            pl.BlockSpec((TM, K), lambda i: (i, 0)),
            pl.BlockSpec((TM, E), lambda i: (i, 0)),
        ],
        out_shape=[
            jax.ShapeDtypeStruct((T, K), jnp.float32),
            jax.ShapeDtypeStruct((T, K), jnp.int32),
            jax.ShapeDtypeStruct((T, E), jnp.float32),
        ],
        compiler_params=pltpu.CompilerParams(
            dimension_semantics=("parallel",)),
        interpret=interpret,
    )(pooled_feat, W1, b1r, W2, b2r)
    return (out[0], out[1], out[2])


# speedup vs baseline: 2.6340x; 1.0026x over previous
import functools

import jax
import jax.numpy as jnp
from jax.experimental import pallas as pl
from jax.experimental.pallas import tpu as pltpu

T = 16384
D = 2048
H = 1024
E = 16
K = 2
TM = 2048
NSTEPS = T // TM


def _tile_outputs(x, w1, b1, w2, b2):
    h = jnp.dot(x, w1, preferred_element_type=jnp.float32)
    h = jnp.maximum(h + b1, 0.0)
    logits = jnp.dot(h, w2, preferred_element_type=jnp.float32)
    logits = logits + b2

    lt = logits.T
    iota = jax.lax.broadcasted_iota(jnp.int32, (E, TM), 0)
    m1 = jnp.max(lt, axis=0, keepdims=True)
    i1 = jnp.min(jnp.where(lt == m1, iota, E), axis=0, keepdims=True)
    masked = jnp.where(iota == i1, -jnp.inf, lt)
    m2 = jnp.max(masked, axis=0, keepdims=True)
    i2 = jnp.min(jnp.where(masked == m2, iota, E), axis=0, keepdims=True)

    et = jnp.exp(lt - m1)
    zt = jnp.sum(et, axis=0, keepdims=True)
    pt = et / zt
    w1p = jnp.max(pt, axis=0, keepdims=True)
    w2p = jnp.max(jnp.where(iota == i1, -1.0, pt), axis=0, keepdims=True)
    denom = jnp.maximum(w1p + w2p, 1e-6)
    wts = jnp.concatenate([w1p, w2p], axis=0) / denom
    idx = jnp.concatenate([i1, i2], axis=0).astype(jnp.float32)

    return wts.T, idx.astype(jnp.int32).T, pt.T


def _router_kernel(x_ref, w1_ref, b1_ref, w2_ref, b2_ref,
                   w_ref, i_ref, p_ref):
    wts, idx, pt = _tile_outputs(x_ref[...], w1_ref[...], b1_ref[...],
                                 w2_ref[...], b2_ref[...])
    w_ref[...] = wts
    i_ref[...] = idx
    p_ref[...] = pt


@functools.partial(jax.jit, static_argnames=("interpret",))
def kernel(pooled_feat, W1, b1, W2, b2, interpret=False):
    b1r = b1.reshape(1, H)
    b2r = b2.reshape(1, E)
    grid = (NSTEPS,)
    out = pl.pallas_call(
        _router_kernel,
        grid=grid,
        in_specs=[
            pl.BlockSpec((TM, D), lambda i: (i, 0)),
            pl.BlockSpec((D, H), lambda i: (0, 0)),
            pl.BlockSpec((1, H), lambda i: (0, 0)),
            pl.BlockSpec((H, E), lambda i: (0, 0)),
            pl.BlockSpec((1, E), lambda i: (0, 0)),
        ],
        out_specs=[
            pl.BlockSpec((TM, K), lambda i: (i, 0)),
            pl.BlockSpec((TM, K), lambda i: (i, 0)),
            pl.BlockSpec((TM, E), lambda i: (i, 0)),
        ],
        out_shape=[
            jax.ShapeDtypeStruct((T, K), jnp.float32),
            jax.ShapeDtypeStruct((T, K), jnp.int32),
            jax.ShapeDtypeStruct((T, E), jnp.float32),
        ],
        compiler_params=pltpu.CompilerParams(
            dimension_semantics=("parallel",)),
        interpret=interpret,
    )(pooled_feat, W1, b1r, W2, b2r)
    return (out[0], out[1], out[2])


# speedup vs baseline: 2.6889x; 1.0209x over previous
import functools

import jax
import jax.numpy as jnp
from jax.experimental import pallas as pl
from jax.experimental.pallas import tpu as pltpu

T = 16384
D = 2048
H = 1024
E = 16
K = 2
TM = 1024
NSTEPS = T // (2 * TM)
TH = T // 2


def _tile_outputs(x, w1, b1, w2, b2):
    h = jnp.dot(x, w1, preferred_element_type=jnp.float32)
    h = jnp.maximum(h + b1, 0.0)
    logits = jnp.dot(h, w2, preferred_element_type=jnp.float32)
    logits = logits + b2

    lt = logits.T
    iota = jax.lax.broadcasted_iota(jnp.int32, (E, TM), 0)
    m1 = jnp.max(lt, axis=0, keepdims=True)
    i1 = jnp.min(jnp.where(lt == m1, iota, E), axis=0, keepdims=True)
    masked = jnp.where(iota == i1, -jnp.inf, lt)
    m2 = jnp.max(masked, axis=0, keepdims=True)
    i2 = jnp.min(jnp.where(masked == m2, iota, E), axis=0, keepdims=True)

    et = jnp.exp(lt - m1)
    zt = jnp.sum(et, axis=0, keepdims=True)
    pt = et / zt
    w1p = jnp.max(pt, axis=0, keepdims=True)
    w2p = jnp.max(jnp.where(iota == i1, -1.0, pt), axis=0, keepdims=True)
    denom = jnp.maximum(w1p + w2p, 1e-6)
    wts = jnp.concatenate([w1p, w2p], axis=0) / denom
    idx = jnp.concatenate([i1, i2], axis=0).astype(jnp.float32)

    return wts.T, idx.astype(jnp.int32).T, pt.T


def _router_kernel(xa_ref, xb_ref, w1_ref, b1_ref, w2_ref, b2_ref,
                   w_ref, i_ref, p_ref):
    w1 = w1_ref[...]
    b1 = b1_ref[...]
    w2 = w2_ref[...]
    b2 = b2_ref[...]
    wa, ia, pa = _tile_outputs(xa_ref[...], w1, b1, w2, b2)
    wb, ib, pb = _tile_outputs(xb_ref[...], w1, b1, w2, b2)
    w_ref[0], w_ref[1] = wa, wb
    i_ref[0], i_ref[1] = ia, ib
    p_ref[0], p_ref[1] = pa, pb


@functools.partial(jax.jit, static_argnames=("interpret",))
def kernel(pooled_feat, W1, b1, W2, b2, interpret=False):
    b1r = b1.reshape(1, H)
    b2r = b2.reshape(1, E)
    grid = (NSTEPS,)
    out = pl.pallas_call(
        _router_kernel,
        grid=grid,
        in_specs=[
            pl.BlockSpec((TM, D), lambda i: (i, 0)),
            pl.BlockSpec((TM, D), lambda i: (i + NSTEPS, 0)),
            pl.BlockSpec((D, H), lambda i: (0, 0)),
            pl.BlockSpec((1, H), lambda i: (0, 0)),
            pl.BlockSpec((H, E), lambda i: (0, 0)),
            pl.BlockSpec((1, E), lambda i: (0, 0)),
        ],
        out_specs=[
            pl.BlockSpec((2, TM, K), lambda i: (0, i, 0)),
            pl.BlockSpec((2, TM, K), lambda i: (0, i, 0)),
            pl.BlockSpec((2, TM, E), lambda i: (0, i, 0)),
        ],
        out_shape=[
            jax.ShapeDtypeStruct((2, TH, K), jnp.float32),
            jax.ShapeDtypeStruct((2, TH, K), jnp.int32),
            jax.ShapeDtypeStruct((2, TH, E), jnp.float32),
        ],
        compiler_params=pltpu.CompilerParams(
            dimension_semantics=("parallel",)),
        interpret=interpret,
    )(pooled_feat, pooled_feat, W1, b1r, W2, b2r)
    return (out[0].reshape(T, K), out[1].reshape(T, K), out[2].reshape(T, E))
